# baseline (speedup 1.0000x reference)
import functools

import jax
import jax.numpy as jnp
from jax import lax
from jax.experimental import pallas as pl
from jax.experimental.pallas import tpu as pltpu
from jax.experimental.pallas import tpu_sc as plsc

N_NODES = 10000
N_EDGES = 160000
LATENT = 128
STEPS = 15
OUT_DIM = 3

EBLK = 640
NBLK = 1000

NC = 2
NS = 16
NW = NC * NS
CH = 128
NCHUNK = N_EDGES // CH
CPW = -(-NCHUNK // NW)

_f32 = jnp.float32


def _dot(a, b):
    return jnp.dot(a, b, preferred_element_type=jnp.float32)


def _ln_tc(d, s, b):
    m = jnp.mean(d, axis=-1, keepdims=True)
    c = d - m
    v = jnp.mean(c * c, axis=-1, keepdims=True)
    return c * lax.rsqrt(v + 1e-5) * s + b


def _enc_body(x_ref, nm_ref, ns_ref, w1_ref, b1_ref, w2_ref, b2_ref,
              w3_ref, b3_ref, out_ref):
    x = (x_ref[...] - nm_ref[...]) / jnp.maximum(ns_ref[...], 1e-8)
    h = jnp.maximum(_dot(x, w1_ref[...]) + b1_ref[...], 0.0)
    h = jnp.maximum(_dot(h, w2_ref[...]) + b2_ref[...], 0.0)
    out_ref[...] = _dot(h, w3_ref[...]) + b3_ref[...]


def _encoder(x, nm, ns, w1, b1, w2, b2, w3, b3, blk):
    n, f = x.shape
    const = lambda i: (0, 0)
    return pl.pallas_call(
        _enc_body,
        grid=(n // blk,),
        in_specs=[
            pl.BlockSpec((blk, f), lambda i: (i, 0)),
            pl.BlockSpec((1, f), const),
            pl.BlockSpec((1, f), const),
            pl.BlockSpec((f, LATENT), const),
            pl.BlockSpec((1, LATENT), const),
            pl.BlockSpec((LATENT, LATENT), const),
            pl.BlockSpec((1, LATENT), const),
            pl.BlockSpec((LATENT, LATENT), const),
            pl.BlockSpec((1, LATENT), const),
        ],
        out_specs=pl.BlockSpec((blk, LATENT), lambda i: (i, 0)),
        out_shape=jax.ShapeDtypeStruct((n, LATENT), _f32),
    )(x, nm.reshape(1, -1), ns.reshape(1, -1), w1, b1.reshape(1, -1), w2,
      b2.reshape(1, -1), w3, b3.reshape(1, -1))


def _cat3_body(a_ref, b_ref, c_ref, w1_ref, b1_ref, w2_ref, b2_ref,
               w3_ref, b3_ref, lns_ref, lnb_ref, out_ref):
    x = jnp.concatenate([a_ref[...], b_ref[...], c_ref[...]], axis=-1)
    h = jnp.maximum(_dot(x, w1_ref[...]) + b1_ref[...], 0.0)
    h = jnp.maximum(_dot(h, w2_ref[...]) + b2_ref[...], 0.0)
    out_ref[...] = _dot(h, w3_ref[...]) + b3_ref[...]


def _cat3_mlp(a, b, c, w1, b1, w2, b2, w3, b3, lns, lnb, blk):
    n = a.shape[0]
    const = lambda i: (0, 0)
    row = lambda i: (i, 0)
    return pl.pallas_call(
        _cat3_body,
        grid=(n // blk,),
        in_specs=[
            pl.BlockSpec((blk, LATENT), row),
            pl.BlockSpec((blk, LATENT), row),
            pl.BlockSpec((blk, LATENT), row),
            pl.BlockSpec((3 * LATENT, LATENT), const),
            pl.BlockSpec((1, LATENT), const),
            pl.BlockSpec((LATENT, LATENT), const),
            pl.BlockSpec((1, LATENT), const),
            pl.BlockSpec((LATENT, LATENT), const),
            pl.BlockSpec((1, LATENT), const),
            pl.BlockSpec((1, LATENT), const),
            pl.BlockSpec((1, LATENT), const),
        ],
        out_specs=pl.BlockSpec((blk, LATENT), row),
        out_shape=jax.ShapeDtypeStruct((n, LATENT), _f32),
    )(a, b, c, w1, b1.reshape(1, -1), w2, b2.reshape(1, -1), w3,
      b3.reshape(1, -1), lns.reshape(1, -1), lnb.reshape(1, -1))


def _cat3_body_bcast(a_ref, b_ref, c_ref, w1_ref, b1_ref, w2_ref, b2_ref,
                     w3_ref, b3_ref, lns_ref, lnb_ref, out_ref):
    blk = a_ref.shape[0]
    gb = jnp.broadcast_to(c_ref[...], (blk, LATENT))
    x = jnp.concatenate([a_ref[...], b_ref[...], gb], axis=-1)
    h = jnp.maximum(_dot(x, w1_ref[...]) + b1_ref[...], 0.0)
    h = jnp.maximum(_dot(h, w2_ref[...]) + b2_ref[...], 0.0)
    out_ref[...] = _dot(h, w3_ref[...]) + b3_ref[...]


def _cat3_mlp_bcast(a, b, g, w1, b1, w2, b2, w3, b3, lns, lnb, blk):
    n = a.shape[0]
    const = lambda i: (0, 0)
    row = lambda i: (i, 0)
    return pl.pallas_call(
        _cat3_body_bcast,
        grid=(n // blk,),
        in_specs=[
            pl.BlockSpec((blk, LATENT), row),
            pl.BlockSpec((blk, LATENT), row),
            pl.BlockSpec((1, LATENT), const),
            pl.BlockSpec((3 * LATENT, LATENT), const),
            pl.BlockSpec((1, LATENT), const),
            pl.BlockSpec((LATENT, LATENT), const),
            pl.BlockSpec((1, LATENT), const),
            pl.BlockSpec((LATENT, LATENT), const),
            pl.BlockSpec((1, LATENT), const),
            pl.BlockSpec((1, LATENT), const),
            pl.BlockSpec((1, LATENT), const),
        ],
        out_specs=pl.BlockSpec((blk, LATENT), row),
        out_shape=jax.ShapeDtypeStruct((n, LATENT), _f32),
    )(a, b, g, w1, b1.reshape(1, -1), w2, b2.reshape(1, -1), w3,
      b3.reshape(1, -1), lns.reshape(1, -1), lnb.reshape(1, -1))


def _dec_body(x_ref, w1_ref, b1_ref, w2_ref, b2_ref, w3_ref, b3_ref, out_ref):
    h = jnp.maximum(_dot(x_ref[...], w1_ref[...]) + b1_ref[...], 0.0)
    h = jnp.maximum(_dot(h, w2_ref[...]) + b2_ref[...], 0.0)
    out_ref[...] = _dot(h, w3_ref[...]) + b3_ref[...]


def _decoder(node, w1, b1, w2, b2, w3, b3):
    const = lambda i: (0, 0)
    w3p = jnp.pad(w3, ((0, 0), (0, LATENT - OUT_DIM)))
    b3p = jnp.pad(b3, (0, LATENT - OUT_DIM))
    out = pl.pallas_call(
        _dec_body,
        grid=(N_NODES // NBLK,),
        in_specs=[
            pl.BlockSpec((NBLK, LATENT), lambda i: (i, 0)),
            pl.BlockSpec((LATENT, LATENT), const),
            pl.BlockSpec((1, LATENT), const),
            pl.BlockSpec((LATENT, LATENT), const),
            pl.BlockSpec((1, LATENT), const),
            pl.BlockSpec((LATENT, LATENT), const),
            pl.BlockSpec((1, LATENT), const),
        ],
        out_specs=pl.BlockSpec((NBLK, LATENT), lambda i: (i, 0)),
        out_shape=jax.ShapeDtypeStruct((N_NODES, LATENT), _f32),
    )(node, w1, b1.reshape(1, -1), w2, b2.reshape(1, -1), w3p,
      b3p.reshape(1, -1))
    return out[:, :OUT_DIM]


@functools.cache
def _build_sc_gather2():
    mesh = plsc.VectorSubcoreMesh(core_axis_name="c", subcore_axis_name="s",
                                  num_cores=NC, num_subcores=NS)

    @functools.partial(
        pl.kernel,
        out_type=(
            jax.ShapeDtypeStruct((N_EDGES, LATENT), _f32),
            jax.ShapeDtypeStruct((N_EDGES, LATENT), _f32),
        ),
        mesh=mesh,
        scratch_types=[
            pltpu.VMEM((CH,), jnp.int32),
            pltpu.VMEM((CH,), jnp.int32),
            pltpu.VMEM((CH, LATENT), _f32),
            pltpu.VMEM((CH, LATENT), _f32),
            pltpu.SemaphoreType.DMA,
            pltpu.SemaphoreType.DMA,
        ],
    )
    def sc_gather2(tab_hbm, snd_hbm, rcv_hbm, ns_out, nr_out,
                   sidx, ridx, sbuf, rbuf, sem1, sem2):
        wid = lax.axis_index("s") * NC + lax.axis_index("c")

        def body(i, carry):
            c = wid + i * NW

            @pl.when(c < NCHUNK)
            def _():
                base = c * CH
                pltpu.sync_copy(snd_hbm.at[pl.ds(base, CH)], sidx)
                pltpu.sync_copy(rcv_hbm.at[pl.ds(base, CH)], ridx)
                cp1 = pltpu.async_copy(tab_hbm.at[sidx], sbuf, sem1)
                cp2 = pltpu.async_copy(tab_hbm.at[ridx], rbuf, sem2)
                cp1.wait()
                cp2.wait()
                pltpu.sync_copy(sbuf, ns_out.at[pl.ds(base, CH)])
                pltpu.sync_copy(rbuf, nr_out.at[pl.ds(base, CH)])

            return carry

        lax.fori_loop(0, CPW, body, 0)

    return sc_gather2


def _sc_gather2(tab, snd, rcv):
    return _build_sc_gather2()(tab, snd, rcv)


def _ln_jnp(d, s, b):
    m = jnp.mean(d, axis=-1, keepdims=True)
    v = jnp.var(d, axis=-1, keepdims=True)
    return (d - m) / jnp.sqrt(v + 1e-5) * s + b


def kernel(node_features, edge_features, global_features, params, senders,
           receivers, is_trainning, prebuild_graph):
    p = params

    node = _encoder(node_features, p['node_norm_mean'], p['node_norm_std'],
                    p['enc_n_w1'], p['enc_n_b1'], p['enc_n_w2'],
                    p['enc_n_b2'], p['enc_n_w3'], p['enc_n_b3'], NBLK)
    node = _ln_jnp(node, p['enc_n_ln_s'], p['enc_n_ln_b'])
    edge = _encoder(edge_features, p['edge_norm_mean'], p['edge_norm_std'],
                    p['enc_e_w1'], p['enc_e_b1'], p['enc_e_w2'],
                    p['enc_e_b2'], p['enc_e_w3'], p['enc_e_b3'], EBLK)
    edge = _ln_jnp(edge, p['enc_e_ln_s'], p['enc_e_ln_b'])
    g = global_features

    for s in range(STEPS):
        pe = 'pe%d' % s
        ns, nr = _sc_gather2(node, senders, receivers)
        d = _cat3_mlp(edge, ns, nr, p[pe + '_w1'], p[pe + '_b1'],
                      p[pe + '_w2'], p[pe + '_b2'], p[pe + '_w3'],
                      p[pe + '_b3'], p[pe + '_ln_s'], p[pe + '_ln_b'],
                      EBLK)
        edge = edge + _ln_jnp(d, p[pe + '_ln_s'], p[pe + '_ln_b'])

        agg = jax.ops.segment_sum(edge, receivers, num_segments=N_NODES)
        pn = 'pn%d' % s
        d = _cat3_mlp_bcast(node, agg, g, p[pn + '_w1'], p[pn + '_b1'],
                            p[pn + '_w2'], p[pn + '_b2'], p[pn + '_w3'],
                            p[pn + '_b3'], p[pn + '_ln_s'],
                            p[pn + '_ln_b'], NBLK)
        node = node + _ln_jnp(d, p[pn + '_ln_s'], p[pn + '_ln_b'])

        pg = 'pg%d' % s
        g_in = jnp.concatenate([g, jnp.mean(node, axis=0, keepdims=True),
                                jnp.mean(edge, axis=0, keepdims=True)],
                               axis=-1)
        h = jnp.maximum(jnp.dot(g_in, p[pg + '_w1']) + p[pg + '_b1'], 0.0)
        h = jnp.maximum(jnp.dot(h, p[pg + '_w2']) + p[pg + '_b2'], 0.0)
        d = jnp.dot(h, p[pg + '_w3']) + p[pg + '_b3']
        g = g + _ln_jnp(d, p[pg + '_ln_s'], p[pg + '_ln_b'])

    return _decoder(node, p['dec_w1'], p['dec_b1'], p['dec_w2'], p['dec_b2'],
                    p['dec_w3'], p['dec_b3'])


# speedup vs baseline: 1.1063x; 1.1063x over previous
import functools

import jax
import jax.numpy as jnp
from jax import lax
from jax.experimental import pallas as pl
from jax.experimental.pallas import tpu as pltpu
from jax.experimental.pallas import tpu_sc as plsc

N_NODES = 10000
N_EDGES = 160000
LATENT = 128
STEPS = 15
OUT_DIM = 3

EBLK = 1600
NBLK = 1000

NC = 2
NS = 16
NW = NC * NS
CH = 128
NCHUNK = N_EDGES // CH
CPW = -(-NCHUNK // NW)

_f32 = jnp.float32


def _dot(a, b):
    return jnp.dot(a, b, preferred_element_type=jnp.float32)


def _ln_tc(d, s, b):
    m = jnp.mean(d, axis=-1, keepdims=True)
    c = d - m
    v = jnp.mean(c * c, axis=-1, keepdims=True)
    return c * lax.rsqrt(v + 1e-5) * s + b


def _enc_body(x_ref, nm_ref, ns_ref, w1_ref, b1_ref, w2_ref, b2_ref,
              w3_ref, b3_ref, out_ref):
    x = (x_ref[...] - nm_ref[...]) / jnp.maximum(ns_ref[...], 1e-8)
    h = jnp.maximum(_dot(x, w1_ref[...]) + b1_ref[...], 0.0)
    h = jnp.maximum(_dot(h, w2_ref[...]) + b2_ref[...], 0.0)
    out_ref[...] = _dot(h, w3_ref[...]) + b3_ref[...]


def _encoder(x, nm, ns, w1, b1, w2, b2, w3, b3, blk):
    n, f = x.shape
    const = lambda i: (0, 0)
    return pl.pallas_call(
        _enc_body,
        grid=(n // blk,),
        in_specs=[
            pl.BlockSpec((blk, f), lambda i: (i, 0)),
            pl.BlockSpec((1, f), const),
            pl.BlockSpec((1, f), const),
            pl.BlockSpec((f, LATENT), const),
            pl.BlockSpec((1, LATENT), const),
            pl.BlockSpec((LATENT, LATENT), const),
            pl.BlockSpec((1, LATENT), const),
            pl.BlockSpec((LATENT, LATENT), const),
            pl.BlockSpec((1, LATENT), const),
        ],
        out_specs=pl.BlockSpec((blk, LATENT), lambda i: (i, 0)),
        out_shape=jax.ShapeDtypeStruct((n, LATENT), _f32),
    )(x, nm.reshape(1, -1), ns.reshape(1, -1), w1, b1.reshape(1, -1), w2,
      b2.reshape(1, -1), w3, b3.reshape(1, -1))


def _cat3_body(a_ref, b_ref, c_ref, w1_ref, b1_ref, w2_ref, b2_ref,
               w3_ref, b3_ref, lns_ref, lnb_ref, out_ref):
    x = jnp.concatenate([a_ref[...], b_ref[...], c_ref[...]], axis=-1)
    h = jnp.maximum(_dot(x, w1_ref[...]) + b1_ref[...], 0.0)
    h = jnp.maximum(_dot(h, w2_ref[...]) + b2_ref[...], 0.0)
    out_ref[...] = _dot(h, w3_ref[...]) + b3_ref[...]


def _cat3_mlp(a, b, c, w1, b1, w2, b2, w3, b3, lns, lnb, blk):
    n = a.shape[0]
    const = lambda i: (0, 0)
    row = lambda i: (i, 0)
    return pl.pallas_call(
        _cat3_body,
        grid=(n // blk,),
        in_specs=[
            pl.BlockSpec((blk, LATENT), row),
            pl.BlockSpec((blk, LATENT), row),
            pl.BlockSpec((blk, LATENT), row),
            pl.BlockSpec((3 * LATENT, LATENT), const),
            pl.BlockSpec((1, LATENT), const),
            pl.BlockSpec((LATENT, LATENT), const),
            pl.BlockSpec((1, LATENT), const),
            pl.BlockSpec((LATENT, LATENT), const),
            pl.BlockSpec((1, LATENT), const),
            pl.BlockSpec((1, LATENT), const),
            pl.BlockSpec((1, LATENT), const),
        ],
        out_specs=pl.BlockSpec((blk, LATENT), row),
        out_shape=jax.ShapeDtypeStruct((n, LATENT), _f32),
    )(a, b, c, w1, b1.reshape(1, -1), w2, b2.reshape(1, -1), w3,
      b3.reshape(1, -1), lns.reshape(1, -1), lnb.reshape(1, -1))


def _cat3_body_bcast(a_ref, b_ref, c_ref, w1_ref, b1_ref, w2_ref, b2_ref,
                     w3_ref, b3_ref, lns_ref, lnb_ref, out_ref):
    blk = a_ref.shape[0]
    gb = jnp.broadcast_to(c_ref[...], (blk, LATENT))
    x = jnp.concatenate([a_ref[...], b_ref[...], gb], axis=-1)
    h = jnp.maximum(_dot(x, w1_ref[...]) + b1_ref[...], 0.0)
    h = jnp.maximum(_dot(h, w2_ref[...]) + b2_ref[...], 0.0)
    out_ref[...] = _dot(h, w3_ref[...]) + b3_ref[...]


def _cat3_mlp_bcast(a, b, g, w1, b1, w2, b2, w3, b3, lns, lnb, blk):
    n = a.shape[0]
    const = lambda i: (0, 0)
    row = lambda i: (i, 0)
    return pl.pallas_call(
        _cat3_body_bcast,
        grid=(n // blk,),
        in_specs=[
            pl.BlockSpec((blk, LATENT), row),
            pl.BlockSpec((blk, LATENT), row),
            pl.BlockSpec((1, LATENT), const),
            pl.BlockSpec((3 * LATENT, LATENT), const),
            pl.BlockSpec((1, LATENT), const),
            pl.BlockSpec((LATENT, LATENT), const),
            pl.BlockSpec((1, LATENT), const),
            pl.BlockSpec((LATENT, LATENT), const),
            pl.BlockSpec((1, LATENT), const),
            pl.BlockSpec((1, LATENT), const),
            pl.BlockSpec((1, LATENT), const),
        ],
        out_specs=pl.BlockSpec((blk, LATENT), row),
        out_shape=jax.ShapeDtypeStruct((n, LATENT), _f32),
    )(a, b, g, w1, b1.reshape(1, -1), w2, b2.reshape(1, -1), w3,
      b3.reshape(1, -1), lns.reshape(1, -1), lnb.reshape(1, -1))


def _dec_body(x_ref, w1_ref, b1_ref, w2_ref, b2_ref, w3_ref, b3_ref, out_ref):
    h = jnp.maximum(_dot(x_ref[...], w1_ref[...]) + b1_ref[...], 0.0)
    h = jnp.maximum(_dot(h, w2_ref[...]) + b2_ref[...], 0.0)
    out_ref[...] = _dot(h, w3_ref[...]) + b3_ref[...]


def _decoder(node, w1, b1, w2, b2, w3, b3):
    const = lambda i: (0, 0)
    w3p = jnp.pad(w3, ((0, 0), (0, LATENT - OUT_DIM)))
    b3p = jnp.pad(b3, (0, LATENT - OUT_DIM))
    out = pl.pallas_call(
        _dec_body,
        grid=(N_NODES // NBLK,),
        in_specs=[
            pl.BlockSpec((NBLK, LATENT), lambda i: (i, 0)),
            pl.BlockSpec((LATENT, LATENT), const),
            pl.BlockSpec((1, LATENT), const),
            pl.BlockSpec((LATENT, LATENT), const),
            pl.BlockSpec((1, LATENT), const),
            pl.BlockSpec((LATENT, LATENT), const),
            pl.BlockSpec((1, LATENT), const),
        ],
        out_specs=pl.BlockSpec((NBLK, LATENT), lambda i: (i, 0)),
        out_shape=jax.ShapeDtypeStruct((N_NODES, LATENT), _f32),
    )(node, w1, b1.reshape(1, -1), w2, b2.reshape(1, -1), w3p,
      b3p.reshape(1, -1))
    return out[:, :OUT_DIM]


@functools.cache
def _build_sc_gather2():
    mesh = plsc.VectorSubcoreMesh(core_axis_name="c", subcore_axis_name="s",
                                  num_cores=NC, num_subcores=NS)

    @functools.partial(
        pl.kernel,
        out_type=(
            jax.ShapeDtypeStruct((N_EDGES, LATENT), _f32),
            jax.ShapeDtypeStruct((N_EDGES, LATENT), _f32),
        ),
        mesh=mesh,
        scratch_types=[
            pltpu.VMEM((CH,), jnp.int32),
            pltpu.VMEM((CH,), jnp.int32),
            pltpu.VMEM((CH, LATENT), _f32),
            pltpu.VMEM((CH, LATENT), _f32),
            pltpu.SemaphoreType.DMA,
            pltpu.SemaphoreType.DMA,
        ],
    )
    def sc_gather2(tab_hbm, snd_hbm, rcv_hbm, ns_out, nr_out,
                   sidx, ridx, sbuf, rbuf, sem1, sem2):
        wid = lax.axis_index("s") * NC + lax.axis_index("c")

        def body(i, carry):
            c = wid + i * NW

            @pl.when(c < NCHUNK)
            def _():
                base = c * CH
                pltpu.sync_copy(snd_hbm.at[pl.ds(base, CH)], sidx)
                pltpu.sync_copy(rcv_hbm.at[pl.ds(base, CH)], ridx)
                cp1 = pltpu.async_copy(tab_hbm.at[sidx], sbuf, sem1)
                cp2 = pltpu.async_copy(tab_hbm.at[ridx], rbuf, sem2)
                cp1.wait()
                cp2.wait()
                pltpu.sync_copy(sbuf, ns_out.at[pl.ds(base, CH)])
                pltpu.sync_copy(rbuf, nr_out.at[pl.ds(base, CH)])

            return carry

        lax.fori_loop(0, CPW, body, 0)

    return sc_gather2


def _sc_gather2(tab, snd, rcv):
    return _build_sc_gather2()(tab, snd, rcv)


def _ln_jnp(d, s, b):
    m = jnp.mean(d, axis=-1, keepdims=True)
    v = jnp.var(d, axis=-1, keepdims=True)
    return (d - m) / jnp.sqrt(v + 1e-5) * s + b


def kernel(node_features, edge_features, global_features, params, senders,
           receivers, is_trainning, prebuild_graph):
    p = params

    node = _encoder(node_features, p['node_norm_mean'], p['node_norm_std'],
                    p['enc_n_w1'], p['enc_n_b1'], p['enc_n_w2'],
                    p['enc_n_b2'], p['enc_n_w3'], p['enc_n_b3'], NBLK)
    node = _ln_jnp(node, p['enc_n_ln_s'], p['enc_n_ln_b'])
    edge = _encoder(edge_features, p['edge_norm_mean'], p['edge_norm_std'],
                    p['enc_e_w1'], p['enc_e_b1'], p['enc_e_w2'],
                    p['enc_e_b2'], p['enc_e_w3'], p['enc_e_b3'], EBLK)
    edge = _ln_jnp(edge, p['enc_e_ln_s'], p['enc_e_ln_b'])
    g = global_features

    for s in range(STEPS):
        pe = 'pe%d' % s
        ns, nr = _sc_gather2(node, senders, receivers)
        d = _cat3_mlp(edge, ns, nr, p[pe + '_w1'], p[pe + '_b1'],
                      p[pe + '_w2'], p[pe + '_b2'], p[pe + '_w3'],
                      p[pe + '_b3'], p[pe + '_ln_s'], p[pe + '_ln_b'],
                      EBLK)
        edge = edge + _ln_jnp(d, p[pe + '_ln_s'], p[pe + '_ln_b'])

        agg = jax.ops.segment_sum(edge, receivers, num_segments=N_NODES)
        pn = 'pn%d' % s
        d = _cat3_mlp_bcast(node, agg, g, p[pn + '_w1'], p[pn + '_b1'],
                            p[pn + '_w2'], p[pn + '_b2'], p[pn + '_w3'],
                            p[pn + '_b3'], p[pn + '_ln_s'],
                            p[pn + '_ln_b'], NBLK)
        node = node + _ln_jnp(d, p[pn + '_ln_s'], p[pn + '_ln_b'])

        pg = 'pg%d' % s
        g_in = jnp.concatenate([g, jnp.mean(node, axis=0, keepdims=True),
                                jnp.mean(edge, axis=0, keepdims=True)],
                               axis=-1)
        h = jnp.maximum(jnp.dot(g_in, p[pg + '_w1']) + p[pg + '_b1'], 0.0)
        h = jnp.maximum(jnp.dot(h, p[pg + '_w2']) + p[pg + '_b2'], 0.0)
        d = jnp.dot(h, p[pg + '_w3']) + p[pg + '_b3']
        g = g + _ln_jnp(d, p[pg + '_ln_s'], p[pg + '_ln_b'])

    return _decoder(node, p['dec_w1'], p['dec_b1'], p['dec_w2'], p['dec_b2'],
                    p['dec_w3'], p['dec_b3'])


# speedup vs baseline: 1.1268x; 1.0185x over previous
import functools

import jax
import jax.numpy as jnp
from jax import lax
from jax.experimental import pallas as pl
from jax.experimental.pallas import tpu as pltpu
from jax.experimental.pallas import tpu_sc as plsc

N_NODES = 10000
N_EDGES = 160000
LATENT = 128
STEPS = 15
OUT_DIM = 3

EBLK = 2000
NBLK = 2000

NC = 2
NS = 16
NW = NC * NS
CH = 128
NCHUNK = N_EDGES // CH
CPW = -(-NCHUNK // NW)

_f32 = jnp.float32


def _dot(a, b):
    return jnp.dot(a, b, preferred_element_type=jnp.float32)


def _ln_tc(d, s, b):
    m = jnp.mean(d, axis=-1, keepdims=True)
    c = d - m
    v = jnp.mean(c * c, axis=-1, keepdims=True)
    return c * lax.rsqrt(v + 1e-5) * s + b


def _enc_body(x_ref, nm_ref, ns_ref, w1_ref, b1_ref, w2_ref, b2_ref,
              w3_ref, b3_ref, out_ref):
    x = (x_ref[...] - nm_ref[...]) / jnp.maximum(ns_ref[...], 1e-8)
    h = jnp.maximum(_dot(x, w1_ref[...]) + b1_ref[...], 0.0)
    h = jnp.maximum(_dot(h, w2_ref[...]) + b2_ref[...], 0.0)
    out_ref[...] = _dot(h, w3_ref[...]) + b3_ref[...]


def _encoder(x, nm, ns, w1, b1, w2, b2, w3, b3, blk):
    n, f = x.shape
    const = lambda i: (0, 0)
    return pl.pallas_call(
        _enc_body,
        grid=(n // blk,),
        in_specs=[
            pl.BlockSpec((blk, f), lambda i: (i, 0)),
            pl.BlockSpec((1, f), const),
            pl.BlockSpec((1, f), const),
            pl.BlockSpec((f, LATENT), const),
            pl.BlockSpec((1, LATENT), const),
            pl.BlockSpec((LATENT, LATENT), const),
            pl.BlockSpec((1, LATENT), const),
            pl.BlockSpec((LATENT, LATENT), const),
            pl.BlockSpec((1, LATENT), const),
        ],
        out_specs=pl.BlockSpec((blk, LATENT), lambda i: (i, 0)),
        out_shape=jax.ShapeDtypeStruct((n, LATENT), _f32),
    )(x, nm.reshape(1, -1), ns.reshape(1, -1), w1, b1.reshape(1, -1), w2,
      b2.reshape(1, -1), w3, b3.reshape(1, -1))


def _cat3_body(a_ref, b_ref, c_ref, w1_ref, b1_ref, w2_ref, b2_ref,
               w3_ref, b3_ref, lns_ref, lnb_ref, out_ref):
    x = jnp.concatenate([a_ref[...], b_ref[...], c_ref[...]], axis=-1)
    h = jnp.maximum(_dot(x, w1_ref[...]) + b1_ref[...], 0.0)
    h = jnp.maximum(_dot(h, w2_ref[...]) + b2_ref[...], 0.0)
    out_ref[...] = _dot(h, w3_ref[...]) + b3_ref[...]


def _cat3_mlp(a, b, c, w1, b1, w2, b2, w3, b3, lns, lnb, blk):
    n = a.shape[0]
    const = lambda i: (0, 0)
    row = lambda i: (i, 0)
    return pl.pallas_call(
        _cat3_body,
        grid=(n // blk,),
        in_specs=[
            pl.BlockSpec((blk, LATENT), row),
            pl.BlockSpec((blk, LATENT), row),
            pl.BlockSpec((blk, LATENT), row),
            pl.BlockSpec((3 * LATENT, LATENT), const),
            pl.BlockSpec((1, LATENT), const),
            pl.BlockSpec((LATENT, LATENT), const),
            pl.BlockSpec((1, LATENT), const),
            pl.BlockSpec((LATENT, LATENT), const),
            pl.BlockSpec((1, LATENT), const),
            pl.BlockSpec((1, LATENT), const),
            pl.BlockSpec((1, LATENT), const),
        ],
        out_specs=pl.BlockSpec((blk, LATENT), row),
        out_shape=jax.ShapeDtypeStruct((n, LATENT), _f32),
    )(a, b, c, w1, b1.reshape(1, -1), w2, b2.reshape(1, -1), w3,
      b3.reshape(1, -1), lns.reshape(1, -1), lnb.reshape(1, -1))


def _cat3_body_bcast(a_ref, b_ref, c_ref, w1_ref, b1_ref, w2_ref, b2_ref,
                     w3_ref, b3_ref, lns_ref, lnb_ref, out_ref):
    blk = a_ref.shape[0]
    gb = jnp.broadcast_to(c_ref[...], (blk, LATENT))
    x = jnp.concatenate([a_ref[...], b_ref[...], gb], axis=-1)
    h = jnp.maximum(_dot(x, w1_ref[...]) + b1_ref[...], 0.0)
    h = jnp.maximum(_dot(h, w2_ref[...]) + b2_ref[...], 0.0)
    out_ref[...] = _dot(h, w3_ref[...]) + b3_ref[...]


def _cat3_mlp_bcast(a, b, g, w1, b1, w2, b2, w3, b3, lns, lnb, blk):
    n = a.shape[0]
    const = lambda i: (0, 0)
    row = lambda i: (i, 0)
    return pl.pallas_call(
        _cat3_body_bcast,
        grid=(n // blk,),
        in_specs=[
            pl.BlockSpec((blk, LATENT), row),
            pl.BlockSpec((blk, LATENT), row),
            pl.BlockSpec((1, LATENT), const),
            pl.BlockSpec((3 * LATENT, LATENT), const),
            pl.BlockSpec((1, LATENT), const),
            pl.BlockSpec((LATENT, LATENT), const),
            pl.BlockSpec((1, LATENT), const),
            pl.BlockSpec((LATENT, LATENT), const),
            pl.BlockSpec((1, LATENT), const),
            pl.BlockSpec((1, LATENT), const),
            pl.BlockSpec((1, LATENT), const),
        ],
        out_specs=pl.BlockSpec((blk, LATENT), row),
        out_shape=jax.ShapeDtypeStruct((n, LATENT), _f32),
    )(a, b, g, w1, b1.reshape(1, -1), w2, b2.reshape(1, -1), w3,
      b3.reshape(1, -1), lns.reshape(1, -1), lnb.reshape(1, -1))


def _dec_body(x_ref, w1_ref, b1_ref, w2_ref, b2_ref, w3_ref, b3_ref, out_ref):
    h = jnp.maximum(_dot(x_ref[...], w1_ref[...]) + b1_ref[...], 0.0)
    h = jnp.maximum(_dot(h, w2_ref[...]) + b2_ref[...], 0.0)
    out_ref[...] = _dot(h, w3_ref[...]) + b3_ref[...]


def _decoder(node, w1, b1, w2, b2, w3, b3):
    const = lambda i: (0, 0)
    w3p = jnp.pad(w3, ((0, 0), (0, LATENT - OUT_DIM)))
    b3p = jnp.pad(b3, (0, LATENT - OUT_DIM))
    out = pl.pallas_call(
        _dec_body,
        grid=(N_NODES // NBLK,),
        in_specs=[
            pl.BlockSpec((NBLK, LATENT), lambda i: (i, 0)),
            pl.BlockSpec((LATENT, LATENT), const),
            pl.BlockSpec((1, LATENT), const),
            pl.BlockSpec((LATENT, LATENT), const),
            pl.BlockSpec((1, LATENT), const),
            pl.BlockSpec((LATENT, LATENT), const),
            pl.BlockSpec((1, LATENT), const),
        ],
        out_specs=pl.BlockSpec((NBLK, LATENT), lambda i: (i, 0)),
        out_shape=jax.ShapeDtypeStruct((N_NODES, LATENT), _f32),
    )(node, w1, b1.reshape(1, -1), w2, b2.reshape(1, -1), w3p,
      b3p.reshape(1, -1))
    return out[:, :OUT_DIM]


@functools.cache
def _build_sc_gather2():
    mesh = plsc.VectorSubcoreMesh(core_axis_name="c", subcore_axis_name="s",
                                  num_cores=NC, num_subcores=NS)

    @functools.partial(
        pl.kernel,
        out_type=(
            jax.ShapeDtypeStruct((N_EDGES, LATENT), _f32),
            jax.ShapeDtypeStruct((N_EDGES, LATENT), _f32),
        ),
        mesh=mesh,
        scratch_types=[
            pltpu.VMEM((CH,), jnp.int32),
            pltpu.VMEM((CH,), jnp.int32),
            pltpu.VMEM((CH, LATENT), _f32),
            pltpu.VMEM((CH, LATENT), _f32),
            pltpu.SemaphoreType.DMA,
            pltpu.SemaphoreType.DMA,
        ],
    )
    def sc_gather2(tab_hbm, snd_hbm, rcv_hbm, ns_out, nr_out,
                   sidx, ridx, sbuf, rbuf, sem1, sem2):
        wid = lax.axis_index("s") * NC + lax.axis_index("c")

        def body(i, carry):
            c = wid + i * NW

            @pl.when(c < NCHUNK)
            def _():
                base = c * CH
                pltpu.sync_copy(snd_hbm.at[pl.ds(base, CH)], sidx)
                pltpu.sync_copy(rcv_hbm.at[pl.ds(base, CH)], ridx)
                cp1 = pltpu.async_copy(tab_hbm.at[sidx], sbuf, sem1)
                cp2 = pltpu.async_copy(tab_hbm.at[ridx], rbuf, sem2)
                cp1.wait()
                cp2.wait()
                pltpu.sync_copy(sbuf, ns_out.at[pl.ds(base, CH)])
                pltpu.sync_copy(rbuf, nr_out.at[pl.ds(base, CH)])

            return carry

        lax.fori_loop(0, CPW, body, 0)

    return sc_gather2


def _sc_gather2(tab, snd, rcv):
    return _build_sc_gather2()(tab, snd, rcv)


def _ln_jnp(d, s, b):
    m = jnp.mean(d, axis=-1, keepdims=True)
    v = jnp.var(d, axis=-1, keepdims=True)
    return (d - m) / jnp.sqrt(v + 1e-5) * s + b


def kernel(node_features, edge_features, global_features, params, senders,
           receivers, is_trainning, prebuild_graph):
    p = params

    node = _encoder(node_features, p['node_norm_mean'], p['node_norm_std'],
                    p['enc_n_w1'], p['enc_n_b1'], p['enc_n_w2'],
                    p['enc_n_b2'], p['enc_n_w3'], p['enc_n_b3'], NBLK)
    node = _ln_jnp(node, p['enc_n_ln_s'], p['enc_n_ln_b'])
    edge = _encoder(edge_features, p['edge_norm_mean'], p['edge_norm_std'],
                    p['enc_e_w1'], p['enc_e_b1'], p['enc_e_w2'],
                    p['enc_e_b2'], p['enc_e_w3'], p['enc_e_b3'], EBLK)
    edge = _ln_jnp(edge, p['enc_e_ln_s'], p['enc_e_ln_b'])
    g = global_features

    for s in range(STEPS):
        pe = 'pe%d' % s
        ns, nr = _sc_gather2(node, senders, receivers)
        d = _cat3_mlp(edge, ns, nr, p[pe + '_w1'], p[pe + '_b1'],
                      p[pe + '_w2'], p[pe + '_b2'], p[pe + '_w3'],
                      p[pe + '_b3'], p[pe + '_ln_s'], p[pe + '_ln_b'],
                      EBLK)
        edge = edge + _ln_jnp(d, p[pe + '_ln_s'], p[pe + '_ln_b'])

        agg = jax.ops.segment_sum(edge, receivers, num_segments=N_NODES)
        pn = 'pn%d' % s
        d = _cat3_mlp_bcast(node, agg, g, p[pn + '_w1'], p[pn + '_b1'],
                            p[pn + '_w2'], p[pn + '_b2'], p[pn + '_w3'],
                            p[pn + '_b3'], p[pn + '_ln_s'],
                            p[pn + '_ln_b'], NBLK)
        node = node + _ln_jnp(d, p[pn + '_ln_s'], p[pn + '_ln_b'])

        pg = 'pg%d' % s
        g_in = jnp.concatenate([g, jnp.mean(node, axis=0, keepdims=True),
                                jnp.mean(edge, axis=0, keepdims=True)],
                               axis=-1)
        h = jnp.maximum(jnp.dot(g_in, p[pg + '_w1']) + p[pg + '_b1'], 0.0)
        h = jnp.maximum(jnp.dot(h, p[pg + '_w2']) + p[pg + '_b2'], 0.0)
        d = jnp.dot(h, p[pg + '_w3']) + p[pg + '_b3']
        g = g + _ln_jnp(d, p[pg + '_ln_s'], p[pg + '_ln_b'])

    return _decoder(node, p['dec_w1'], p['dec_b1'], p['dec_w2'], p['dec_b2'],
                    p['dec_w3'], p['dec_b3'])
